# baseline (speedup 1.0000x reference)
import functools

import numpy as np
import jax
import jax.numpy as jnp
from jax import lax
from jax.experimental import pallas as pl
from jax.experimental.pallas import tpu as pltpu
from jax.experimental.pallas import tpu_sc as plsc

VOCAB_SIZE = 100000
EMB_DIM = 1024
SEQ_LEN = 2048
NUM_SEQ = 4
SCALE = 32.0

_NUM_CORES = 2
_NUM_SUBCORES = 16
_LANES = 16
_NW = _NUM_CORES * _NUM_SUBCORES
_B = NUM_SEQ * SEQ_LEN
_ROWS_PER_W = _B // _NW
_CHUNK = 32
_N_CHUNKS = _ROWS_PER_W // _CHUNK


def _make_pos_encoding(length, depth):
    half = depth / 2
    positions = np.reshape(np.arange(length), [-1, 1])
    depths = np.expand_dims(np.arange(half), axis=0) / half
    angle_rads = positions * (1 / 10000 ** depths)
    return np.concatenate(
        [np.sin(angle_rads), np.cos(angle_rads)], axis=-1
    ).astype(np.float32)


_PE = _make_pos_encoding(SEQ_LEN, EMB_DIM)

_mesh = plsc.VectorSubcoreMesh(core_axis_name="c", subcore_axis_name="s")


@functools.partial(
    pl.kernel,
    mesh=_mesh,
    out_type=jax.ShapeDtypeStruct((_B, EMB_DIM), jnp.float32),
    scratch_types=[
        pltpu.VMEM((_CHUNK,), jnp.int32),
        pltpu.VMEM((_CHUNK, EMB_DIM), jnp.float32),
        pltpu.VMEM((_CHUNK, EMB_DIM), jnp.float32),
        pltpu.SemaphoreType.DMA,
    ],
)
def _sc_embed(table_hbm, idx_hbm, pe_hbm, out_hbm, idx_v, rows_v, pe_v, sem):
    wid = lax.axis_index("s") * _NUM_CORES + lax.axis_index("c")
    base = wid * _ROWS_PER_W
    pe_base = lax.rem(base, SEQ_LEN)

    def chunk_body(c, carry):
        row0 = base + c * _CHUNK
        pltpu.sync_copy(idx_hbm.at[pl.ds(row0, _CHUNK)], idx_v)
        gather = pltpu.async_copy(table_hbm.at[idx_v], rows_v, sem)
        pltpu.sync_copy(pe_hbm.at[pl.ds(pe_base + c * _CHUNK, _CHUNK)], pe_v)
        gather.wait()

        def row_body(r, rc):
            for j in range(EMB_DIM // _LANES):
                sl = pl.ds(j * _LANES, _LANES)
                rows_v[r, sl] = rows_v[r, sl] * SCALE + pe_v[r, sl]
            return rc

        lax.fori_loop(0, _CHUNK, row_body, 0)
        pltpu.sync_copy(rows_v, out_hbm.at[pl.ds(row0, _CHUNK)])
        return carry

    lax.fori_loop(0, _N_CHUNKS, chunk_body, 0)


def kernel(x, table):
    idx = x.reshape(-1).astype(jnp.int32)
    pe = jnp.asarray(_PE)
    out = _sc_embed(table, idx, pe)
    return out.reshape(NUM_SEQ, SEQ_LEN, EMB_DIM)


# speedup vs baseline: 1.4098x; 1.4098x over previous
import functools

import numpy as np
import jax
import jax.numpy as jnp
from jax import lax
from jax.experimental import pallas as pl
from jax.experimental.pallas import tpu as pltpu
from jax.experimental.pallas import tpu_sc as plsc

VOCAB_SIZE = 100000
EMB_DIM = 1024
SEQ_LEN = 2048
NUM_SEQ = 4
SCALE = 32.0

_NUM_CORES = 2
_NUM_SUBCORES = 16
_LANES = 16
_NW = _NUM_CORES * _NUM_SUBCORES
_B = NUM_SEQ * SEQ_LEN
_ROWS_PER_W = _B // _NW
_CHUNK = 16
_N_CHUNKS = _ROWS_PER_W // _CHUNK


def _make_pos_encoding(length, depth):
    half = depth / 2
    positions = np.reshape(np.arange(length), [-1, 1])
    depths = np.expand_dims(np.arange(half), axis=0) / half
    angle_rads = positions * (1 / 10000 ** depths)
    return np.concatenate(
        [np.sin(angle_rads), np.cos(angle_rads)], axis=-1
    ).astype(np.float32)


_PE = _make_pos_encoding(SEQ_LEN, EMB_DIM)

_mesh = plsc.VectorSubcoreMesh(core_axis_name="c", subcore_axis_name="s")


@functools.partial(
    pl.kernel,
    mesh=_mesh,
    out_type=jax.ShapeDtypeStruct((_B, EMB_DIM), jnp.float32),
    scratch_types=[
        pltpu.VMEM((_CHUNK,), jnp.int32),
        pltpu.VMEM((_CHUNK,), jnp.int32),
        pltpu.VMEM((_CHUNK, EMB_DIM), jnp.float32),
        pltpu.VMEM((_CHUNK, EMB_DIM), jnp.float32),
        pltpu.VMEM((_CHUNK, EMB_DIM), jnp.float32),
        pltpu.VMEM((_CHUNK, EMB_DIM), jnp.float32),
        pltpu.VMEM((_CHUNK, EMB_DIM), jnp.float32),
        pltpu.VMEM((_CHUNK, EMB_DIM), jnp.float32),
        pltpu.SemaphoreType.DMA,
        pltpu.SemaphoreType.DMA,
        pltpu.SemaphoreType.DMA,
        pltpu.SemaphoreType.DMA,
        pltpu.SemaphoreType.DMA,
        pltpu.SemaphoreType.DMA,
    ],
)
def _sc_embed(table_hbm, idx_hbm, pe_hbm, out_hbm,
              ix0, ix1, in0, in1, ot0, ot1, pe0, pe1,
              g0, g1, q0, q1, s0, s1):
    idxs, ins, outs, pes = (ix0, ix1), (in0, in1), (ot0, ot1), (pe0, pe1)
    gsem, psem, ssem = (g0, g1), (q0, q1), (s0, s1)

    wid = lax.axis_index("s") * _NUM_CORES + lax.axis_index("c")
    base = wid * _ROWS_PER_W
    pe_base = lax.rem(base, SEQ_LEN)

    def issue(c, k):
        off = c * _CHUNK
        pltpu.sync_copy(idx_hbm.at[pl.ds(base + off, _CHUNK)], idxs[k])
        pltpu.async_copy(table_hbm.at[idxs[k]], ins[k], gsem[k])
        pltpu.async_copy(
            pe_hbm.at[pl.ds(pe_base + off, _CHUNK)], pes[k], psem[k])

    issue(0, 0)
    issue(1, 1)

    def loop_body(i, carry):
        for k in range(2):
            c = 2 * i + k
            pltpu.make_async_copy(
                table_hbm.at[idxs[k]], ins[k], gsem[k]).wait()
            pltpu.make_async_copy(
                pe_hbm.at[pl.ds(0, _CHUNK)], pes[k], psem[k]).wait()

            @pl.when(i >= 1)
            def _wait_store():
                pltpu.make_async_copy(
                    outs[k], out_hbm.at[pl.ds(0, _CHUNK)], ssem[k]).wait()

            def row_body(r, rc):
                for j in range(EMB_DIM // _LANES):
                    sl = pl.ds(j * _LANES, _LANES)
                    outs[k][r, sl] = ins[k][r, sl] * SCALE + pes[k][r, sl]
                return rc

            lax.fori_loop(0, _CHUNK, row_body, 0)

            pltpu.async_copy(
                outs[k], out_hbm.at[pl.ds(base + c * _CHUNK, _CHUNK)],
                ssem[k])

            @pl.when(c + 2 < _N_CHUNKS)
            def _issue_next():
                issue(c + 2, k)
        return carry

    lax.fori_loop(0, _N_CHUNKS // 2, loop_body, 0)

    for k in range(2):
        pltpu.make_async_copy(
            outs[k], out_hbm.at[pl.ds(0, _CHUNK)], ssem[k]).wait()


def kernel(x, table):
    idx = x.reshape(-1).astype(jnp.int32)
    pe = jnp.asarray(_PE)
    out = _sc_embed(table, idx, pe)
    return out.reshape(NUM_SEQ, SEQ_LEN, EMB_DIM)


# speedup vs baseline: 1.4533x; 1.0309x over previous
import functools

import numpy as np
import jax
import jax.numpy as jnp
from jax import lax
from jax.experimental import pallas as pl
from jax.experimental.pallas import tpu as pltpu
from jax.experimental.pallas import tpu_sc as plsc

VOCAB_SIZE = 100000
EMB_DIM = 1024
SEQ_LEN = 2048
NUM_SEQ = 4
SCALE = 32.0

_NUM_CORES = 2
_NUM_SUBCORES = 16
_LANES = 16
_NW = _NUM_CORES * _NUM_SUBCORES
_B = NUM_SEQ * SEQ_LEN
_ROWS_PER_W = _B // _NW
_CHUNK = 16
_N_CHUNKS = _ROWS_PER_W // _CHUNK


def _make_pos_encoding(length, depth):
    half = depth / 2
    positions = np.reshape(np.arange(length), [-1, 1])
    depths = np.expand_dims(np.arange(half), axis=0) / half
    angle_rads = positions * (1 / 10000 ** depths)
    return np.concatenate(
        [np.sin(angle_rads), np.cos(angle_rads)], axis=-1
    ).astype(np.float32)


_PE = _make_pos_encoding(SEQ_LEN, EMB_DIM)

_mesh = plsc.VectorSubcoreMesh(core_axis_name="c", subcore_axis_name="s")


@functools.partial(
    pl.kernel,
    mesh=_mesh,
    out_type=jax.ShapeDtypeStruct((_B, EMB_DIM), jnp.float32),
    scratch_types=[
        pltpu.VMEM((_ROWS_PER_W,), jnp.int32),
        pltpu.VMEM((_CHUNK, EMB_DIM), jnp.float32),
        pltpu.VMEM((_CHUNK, EMB_DIM), jnp.float32),
        pltpu.VMEM((_CHUNK, EMB_DIM), jnp.float32),
        pltpu.VMEM((_CHUNK, EMB_DIM), jnp.float32),
        pltpu.VMEM((_CHUNK, EMB_DIM), jnp.float32),
        pltpu.VMEM((_CHUNK, EMB_DIM), jnp.float32),
        pltpu.SemaphoreType.DMA,
        pltpu.SemaphoreType.DMA,
        pltpu.SemaphoreType.DMA,
        pltpu.SemaphoreType.DMA,
        pltpu.SemaphoreType.DMA,
        pltpu.SemaphoreType.DMA,
    ],
)
def _sc_embed(table_hbm, idx_hbm, pe_hbm, out_hbm,
              idx_v, in0, in1, ot0, ot1, pe0, pe1,
              g0, g1, q0, q1, s0, s1):
    ins, outs, pes = (in0, in1), (ot0, ot1), (pe0, pe1)
    gsem, psem, ssem = (g0, g1), (q0, q1), (s0, s1)

    wid = lax.axis_index("s") * _NUM_CORES + lax.axis_index("c")
    base = wid * _ROWS_PER_W
    pe_base = lax.rem(base, SEQ_LEN)

    pltpu.sync_copy(idx_hbm.at[pl.ds(base, _ROWS_PER_W)], idx_v)

    def issue(c, k):
        off = c * _CHUNK
        idx_vec = idx_v[pl.ds(off, _CHUNK)]
        pltpu.async_copy(table_hbm.at[idx_vec], ins[k], gsem[k])
        pltpu.async_copy(
            pe_hbm.at[pl.ds(pe_base + off, _CHUNK)], pes[k], psem[k])

    issue(0, 0)
    issue(1, 1)

    def loop_body(i, carry):
        for k in range(2):
            c = 2 * i + k
            pltpu.make_async_copy(
                table_hbm.at[idx_v[pl.ds(0, _CHUNK)]], ins[k],
                gsem[k]).wait()
            pltpu.make_async_copy(
                pe_hbm.at[pl.ds(0, _CHUNK)], pes[k], psem[k]).wait()

            @pl.when(i >= 1)
            def _wait_store():
                pltpu.make_async_copy(
                    outs[k], out_hbm.at[pl.ds(0, _CHUNK)], ssem[k]).wait()

            def row_body(r, rc):
                for j in range(EMB_DIM // _LANES):
                    sl = pl.ds(j * _LANES, _LANES)
                    outs[k][r, sl] = ins[k][r, sl] * SCALE + pes[k][r, sl]
                return rc

            lax.fori_loop(0, _CHUNK, row_body, 0)

            pltpu.async_copy(
                outs[k], out_hbm.at[pl.ds(base + c * _CHUNK, _CHUNK)],
                ssem[k])

            @pl.when(c + 2 < _N_CHUNKS)
            def _issue_next():
                issue(c + 2, k)
        return carry

    lax.fori_loop(0, _N_CHUNKS // 2, loop_body, 0)

    for k in range(2):
        pltpu.make_async_copy(
            outs[k], out_hbm.at[pl.ds(0, _CHUNK)], ssem[k]).wait()


def kernel(x, table):
    idx = x.reshape(-1).astype(jnp.int32)
    pe = jnp.asarray(_PE)
    out = _sc_embed(table, idx, pe)
    return out.reshape(NUM_SEQ, SEQ_LEN, EMB_DIM)
